# baseline (speedup 1.0000x reference)
import jax
import jax.numpy as jnp
from jax.experimental import pallas as pl
from jax.experimental.pallas import tpu as pltpu

NBUF = 4
CHUNK_V = 2048


def _gather_body(idx_ref, embed_ref, out_ref):
    out_ref[...] = embed_ref[...]


def _matmul_body(h_ref, w_hbm, o_ref, bufs, sems, tail_buf, tail_sem):
    v = w_hbm.shape[1]
    v_aligned = (v // 128) * 128
    tail_w = v - v_aligned
    chunks = []
    off = 0
    while off < v_aligned:
        w = min(CHUNK_V, v_aligned - off)
        chunks.append((off, w))
        off += w

    def copy(i):
        off, w = chunks[i]
        return pltpu.make_async_copy(
            w_hbm.at[:, pl.ds(off, w)],
            bufs.at[i % NBUF, :, pl.ds(0, w)],
            sems.at[i % NBUF],
        )

    tail_copy = None
    if tail_w:
        tail_copy = pltpu.make_async_copy(
            w_hbm.at[:, pl.ds(v_aligned, tail_w)], tail_buf, tail_sem)
        tail_copy.start()
    for i in range(min(NBUF, len(chunks))):
        copy(i).start()
    for i, (off, w) in enumerate(chunks):
        copy(i).wait()
        o_ref[:, off:off + w] = jnp.dot(
            h_ref[...], bufs[i % NBUF, :, :w],
            preferred_element_type=jnp.float32)
        if i + NBUF < len(chunks):
            copy(i + NBUF).start()
    if tail_w:
        tail_copy.wait()
        o_ref[:, v_aligned:v] = jnp.dot(
            h_ref[...], tail_buf[...], preferred_element_type=jnp.float32)


@jax.jit
def kernel(x, embed, w_out):
    b, s = x.shape
    n_tok = b * s
    vocab = w_out.shape[1]
    d = embed.shape[1]
    idx = x.reshape(n_tok)

    h = jnp.take(embed, idx, axis=0)

    logits = pl.pallas_call(
        _matmul_body,
        in_specs=[
            pl.BlockSpec(memory_space=pltpu.VMEM),
            pl.BlockSpec(memory_space=pl.ANY),
        ],
        out_specs=pl.BlockSpec(memory_space=pltpu.VMEM),
        out_shape=jax.ShapeDtypeStruct((n_tok, vocab), jnp.float32),
        scratch_shapes=[
            pltpu.VMEM((NBUF, d, CHUNK_V), jnp.float32),
            pltpu.SemaphoreType.DMA((NBUF,)),
            pltpu.VMEM((d, vocab - (vocab // 128) * 128), jnp.float32),
            pltpu.SemaphoreType.DMA,
        ],
    )(h, w_out)

    return logits.reshape(b, s, vocab)
